# Initial kernel scaffold; baseline (speedup 1.0000x reference)
import jax
import jax.numpy as jnp
from jax.experimental import pallas as pl


def kernel(x, position_weight, value_weight):
    raise NotImplementedError("write your pallas kernel here")


# speedup vs baseline: 7.0673x; 7.0673x over previous
import jax
import jax.numpy as jnp
from jax.experimental import pallas as pl


def _enc_kernel(x_ref, pos_ref, val_ref, out_ref):
    P = x_ref.shape[-1]
    L = val_ref.shape[0]
    xf = x_ref[0, 0, :].astype(jnp.float32)
    idx = jnp.round(xf * (255.0 / 256.0))
    idx = jnp.clip(idx, 0.0, 255.0).astype(jnp.int32)
    lvl = jax.lax.broadcasted_iota(jnp.int32, (L, P), 0)
    onehot = (lvl == idx[None, :]).astype(jnp.bfloat16)
    h = jnp.dot(onehot, pos_ref[...], preferred_element_type=jnp.float32)
    s = jnp.sum(h * val_ref[...], axis=0, keepdims=True)
    out_ref[...] = jnp.where(s > 0, jnp.float32(1.0), jnp.float32(-1.0))[None]


@jax.jit
def kernel(x, position_weight, value_weight):
    B = x.shape[0]
    P = x.shape[1] * x.shape[2]
    L, D = value_weight.shape
    flat = x.reshape(B, 1, P)
    pos_bf = position_weight.astype(jnp.bfloat16)
    out = pl.pallas_call(
        _enc_kernel,
        grid=(B,),
        in_specs=[
            pl.BlockSpec((1, 1, P), lambda b: (b, 0, 0)),
            pl.BlockSpec((P, D), lambda b: (0, 0)),
            pl.BlockSpec((L, D), lambda b: (0, 0)),
        ],
        out_specs=pl.BlockSpec((1, 1, D), lambda b: (b, 0, 0)),
        out_shape=jax.ShapeDtypeStruct((B, 1, D), jnp.float32),
    )(flat, pos_bf, value_weight)
    return out.reshape(B, D)


# speedup vs baseline: 7.2792x; 1.0300x over previous
import jax
import jax.numpy as jnp
from jax.experimental import pallas as pl


_BB = 8


def _enc_kernel(x_ref, pos_ref, val_ref, out_ref):
    BB = x_ref.shape[0]
    P = x_ref.shape[-1]
    L = val_ref.shape[0]
    xf = x_ref[:, 0, :].astype(jnp.float32)
    idx = jnp.round(xf * (255.0 / 256.0))
    idx = jnp.clip(idx, 0.0, 255.0).astype(jnp.int32)
    lvl = jax.lax.broadcasted_iota(jnp.int32, (BB, L, P), 1)
    onehot = (lvl == idx[:, None, :]).astype(jnp.bfloat16)
    onehot = onehot.reshape(BB * L, P)
    h = jnp.dot(onehot, pos_ref[...], preferred_element_type=jnp.float32)
    h = h.reshape(BB, L, -1)
    s = jnp.sum(h * val_ref[...][None], axis=1)
    out_ref[...] = jnp.where(s > 0, jnp.float32(1.0), jnp.float32(-1.0))[:, None, :]


@jax.jit
def kernel(x, position_weight, value_weight):
    B = x.shape[0]
    P = x.shape[1] * x.shape[2]
    L, D = value_weight.shape
    flat = x.reshape(B, 1, P)
    pos_bf = position_weight.astype(jnp.bfloat16)
    out = pl.pallas_call(
        _enc_kernel,
        grid=(B // _BB,),
        in_specs=[
            pl.BlockSpec((_BB, 1, P), lambda b: (b, 0, 0)),
            pl.BlockSpec((P, D), lambda b: (0, 0)),
            pl.BlockSpec((L, D), lambda b: (0, 0)),
        ],
        out_specs=pl.BlockSpec((_BB, 1, D), lambda b: (b, 0, 0)),
        out_shape=jax.ShapeDtypeStruct((B, 1, D), jnp.float32),
    )(flat, pos_bf, value_weight)
    return out.reshape(B, D)
